# baseline (speedup 1.0000x reference)
import jax
import jax.numpy as jnp
from jax import lax
from jax.experimental import pallas as pl
from jax.experimental.pallas import tpu as pltpu

_TOPK = 2
_E = 16


def _router_body(x_ref, w_ref, b_ref, topw_ref, topi_ref, rw_ref):
    xb = x_ref[...]
    w = w_ref[...]
    b = b_ref[...]
    logits = lax.dot_general(
        xb, w, (((1,), (1,)), ((), ())),
        preferred_element_type=jnp.float32) + b
    probs = 1.0 / (1.0 + jnp.exp(-logits))
    eidx = lax.broadcasted_iota(jnp.int32, probs.shape, 1).astype(jnp.float32)
    max1 = jnp.max(probs, axis=1, keepdims=True)
    idx1 = jnp.min(jnp.where(probs == max1, eidx, float(_E)), axis=1,
                   keepdims=True)
    probs_m = jnp.where(eidx == idx1, -jnp.inf, probs)
    max2 = jnp.max(probs_m, axis=1, keepdims=True)
    idx2 = jnp.min(jnp.where(probs_m == max2, eidx, float(_E)), axis=1,
                   keepdims=True)
    topw_ref[...] = jnp.concatenate([max1, max2], axis=1)
    topi_ref[...] = jnp.concatenate([idx1, idx2], axis=1).astype(jnp.int32)
    keep = (eidx == idx1) | (eidx == idx2)
    rw_ref[...] = jnp.where(keep, probs, 0.0)


def kernel(x, W, b):
    batch, seq, d = x.shape
    n = batch * seq
    xf = x.reshape(n, d)
    bt = 1024
    grid = (n // bt,)
    out_shapes = (
        jax.ShapeDtypeStruct((n, _TOPK), jnp.float32),
        jax.ShapeDtypeStruct((n, _TOPK), jnp.int32),
        jax.ShapeDtypeStruct((n, _E), jnp.float32),
    )
    topw, topi, rw = pl.pallas_call(
        _router_body,
        grid=grid,
        in_specs=[
            pl.BlockSpec((bt, d), lambda i: (i, 0)),
            pl.BlockSpec((_E, d), lambda i: (0, 0)),
            pl.BlockSpec((1, _E), lambda i: (0, 0)),
        ],
        out_specs=[
            pl.BlockSpec((bt, _TOPK), lambda i: (i, 0)),
            pl.BlockSpec((bt, _TOPK), lambda i: (i, 0)),
            pl.BlockSpec((bt, _E), lambda i: (i, 0)),
        ],
        out_shape=out_shapes,
        compiler_params=pltpu.CompilerParams(
            vmem_limit_bytes=100 * 1024 * 1024),
    )(xf, W, b.reshape(1, _E))
    return topw, topi, rw
